# baseline (speedup 1.0000x reference)
import functools

import jax
import jax.numpy as jnp
from jax import lax
from jax.experimental import pallas as pl
from jax.experimental.pallas import tpu as pltpu
from jax.experimental.pallas import tpu_sc as plsc

_NBUF = 4
_DIST = 2
_NW = 32


def _format_kernel(D, V):
  mesh = plsc.VectorSubcoreMesh(core_axis_name="c", subcore_axis_name="s")
  n_rt = V // 128
  res = V - n_rt * 128

  @functools.partial(
      pl.kernel,
      mesh=mesh,
      out_type=jax.ShapeDtypeStruct((V * D,), jnp.float32),
      scratch_types=[
          [pltpu.VMEM((D, 128), jnp.float32)] * 2,
          [pltpu.VMEM((128 * D,), jnp.float32)] * 2,
          [pltpu.SemaphoreType.DMA] * 2,
          [pltpu.SemaphoreType.DMA] * 2,
      ],
      compiler_params=pltpu.CompilerParams(use_tc_tiling_on_sc=True,
                                           needs_layout_passes=False),
  )
  def k(tT_hbm, resid_hbm, out_hbm, ins, outs, isems, osems):
    nc = lax.axis_size("c")
    wid = lax.axis_index("s") * nc + lax.axis_index("c")
    per = n_rt // _NW
    rem = n_rt % _NW
    start = wid * per + jnp.minimum(wid, rem)
    count = per + jnp.where(wid < rem, 1, 0)

    lanes = lax.iota(jnp.int32, 16)
    rows_kk = [kk * 16 + lanes for kk in range(4)]

    def in_start(t, b):
      pltpu.async_copy(tT_hbm.at[:, pl.ds((start + t) * 128, 128)], ins[b],
                       isems[b])

    def in_wait(t, b):
      pltpu.make_async_copy(tT_hbm.at[:, pl.ds((start + t) * 128, 128)],
                            ins[b], isems[b]).wait()

    def out_start(t, b):
      pltpu.async_copy(outs[b],
                       out_hbm.at[pl.ds((start + t) * (128 * D), 128 * D)],
                       osems[b])

    def out_wait(t, b):
      pltpu.make_async_copy(
          outs[b], out_hbm.at[pl.ds((start + t) * (128 * D), 128 * D)],
          osems[b]).wait()

    def transpose(b):
      @pl.loop(0, 128, unroll=2)
      def _row(rr):
        colv = rr + 0 * lanes
        for kk in range(4):
          vals = plsc.load_gather(ins[b], [rows_kk[kk], colv])
          outs[b][pl.ds(rr * D + kk * 16, 16)] = vals

    nfull = per // 2

    in_start(0, 0)
    in_start(1, 1)

    for b in range(2):
      in_wait(b, b)
      transpose(b)
      out_start(b, b)
      in_start(b + 2, b)

    @pl.loop(1, nfull)
    def _body(g):
      for b in range(2):
        t = g * 2 + b
        in_wait(t, b)
        out_wait(t - 2, b)
        transpose(b)
        out_start(t, b)

        @pl.when(t + 2 < count)
        def _():
          in_start(t + 2, b)

    @pl.when(count > per)
    def _():
      in_wait(per, 0)
      out_wait(per - 2, 0)
      transpose(0)
      out_start(per, 0)
      out_wait(per, 0)
      out_wait(per - 1, 1)

    @pl.when(count <= per)
    def _():
      out_wait(per - 2, 0)
      out_wait(per - 1, 1)

    if res:
      @pl.when(wid == _NW - 1)
      def _():
        pltpu.sync_copy(resid_hbm, outs[0].at[pl.ds(0, res * D)])
        pltpu.sync_copy(outs[0].at[pl.ds(0, res * D)],
                        out_hbm.at[pl.ds(n_rt * 128 * D, res * D)])

  return k


def _gather_kernel(B, D, n_per_w, chunk):
  mesh = plsc.VectorSubcoreMesh(core_axis_name="c", subcore_axis_name="s")
  n_chunks = n_per_w // chunk
  n_groups = n_chunks // _NBUF
  assert n_groups >= 2 and n_chunks % _NBUF == 0

  @functools.partial(
      pl.kernel,
      mesh=mesh,
      out_type=jax.ShapeDtypeStruct((B, 128), jnp.float32),
      scratch_types=[
          pltpu.VMEM((n_per_w,), jnp.int32),
          [pltpu.VMEM((chunk, D), jnp.float32)] * _NBUF,
          [pltpu.SemaphoreType.DMA] * _NBUF,
          [pltpu.SemaphoreType.DMA] * _NBUF,
      ],
      compiler_params=pltpu.CompilerParams(use_tc_tiling_on_sc=False),
  )
  def k(idx_hbm, table_hbm, out_hbm, idx_v, rows, gsems, wsems):
    nc = lax.axis_size("c")
    wid = lax.axis_index("s") * nc + lax.axis_index("c")
    base = wid * n_per_w
    pltpu.sync_copy(idx_hbm.at[pl.ds(base, n_per_w)], idx_v)

    def g_start(i, b):
      pltpu.async_copy(table_hbm.at[idx_v.at[pl.ds(i * chunk, chunk)]],
                       rows[b], gsems[b])

    def g_wait(i, b):
      pltpu.make_async_copy(table_hbm.at[idx_v.at[pl.ds(i * chunk, chunk)]],
                            rows[b], gsems[b]).wait()

    def w_start(i, b):
      pltpu.async_copy(rows[b],
                       out_hbm.at[pl.ds(base + i * chunk, chunk), pl.ds(0, D)],
                       wsems[b])

    def w_wait(i, b):
      pltpu.make_async_copy(
          rows[b],
          out_hbm.at[pl.ds(base + i * chunk, chunk), pl.ds(0, D)],
          wsems[b]).wait()

    for b in range(_DIST):
      g_start(b, b)

    for b in range(_NBUF):
      i = b
      g_wait(i, b)
      w_start(i, b)
      bj = (b + _DIST) % _NBUF
      if b >= _NBUF - _DIST:
        w_wait(i + _DIST - _NBUF, bj)
      g_start(i + _DIST, bj)

    @pl.loop(1, n_groups - 1)
    def _g(g):
      for b in range(_NBUF):
        i = g * _NBUF + b
        g_wait(i, b)
        w_start(i, b)
        bj = (b + _DIST) % _NBUF
        w_wait(i + _DIST - _NBUF, bj)
        g_start(i + _DIST, bj)

    tail = []
    for b in range(_NBUF):
      i = n_chunks - _NBUF + b
      g_wait(i, b)
      w_start(i, b)
      bj = (b + _DIST) % _NBUF
      w_wait(i + _DIST - _NBUF, bj)
      if b < _NBUF - _DIST:
        g_start(i + _DIST, bj)
      else:
        tail.append((i, b))
    for i, b in tail:
      w_wait(i, b)

  return k


def kernel(x, emb_weight):
  B0, B1 = x.shape
  V, D = emb_weight.shape
  B = B0 * B1
  n_per_w = B // _NW
  chunk = 200

  idx = x.reshape(B).astype(jnp.int32)
  n_rt = V // 128
  resid = emb_weight[n_rt * 128:].reshape((V - n_rt * 128) * D)
  table = _format_kernel(D, V)(emb_weight.T, resid).reshape(V, D)
  out = _gather_kernel(B, D, n_per_w, chunk)(idx, table)
  return out[:, :D].reshape(B0, B1, D)


# speedup vs baseline: 1.9669x; 1.9669x over previous
import functools

import jax
import jax.numpy as jnp
from jax import lax
from jax.experimental import pallas as pl
from jax.experimental.pallas import tpu as pltpu
from jax.experimental.pallas import tpu_sc as plsc

_NBUF = 4
_DIST = 2
_NW = 32


def _gather_kernel(B, D, n_per_w, chunk):
  mesh = plsc.VectorSubcoreMesh(core_axis_name="c", subcore_axis_name="s")
  n_chunks = n_per_w // chunk
  n_groups = n_chunks // _NBUF
  assert n_groups >= 2 and n_chunks % _NBUF == 0

  @functools.partial(
      pl.kernel,
      mesh=mesh,
      out_type=jax.ShapeDtypeStruct((B, 128), jnp.float32),
      scratch_types=[
          pltpu.VMEM((n_per_w,), jnp.int32),
          [pltpu.VMEM((chunk, D), jnp.float32)] * _NBUF,
          [pltpu.SemaphoreType.DMA] * _NBUF,
          [pltpu.SemaphoreType.DMA] * _NBUF,
      ],
      compiler_params=pltpu.CompilerParams(use_tc_tiling_on_sc=False),
  )
  def k(idx_hbm, table_hbm, out_hbm, idx_v, rows, gsems, wsems):
    nc = lax.axis_size("c")
    wid = lax.axis_index("s") * nc + lax.axis_index("c")
    base = wid * n_per_w
    pltpu.sync_copy(idx_hbm.at[pl.ds(base, n_per_w)], idx_v)

    def g_start(i, b):
      pltpu.async_copy(table_hbm.at[idx_v.at[pl.ds(i * chunk, chunk)]],
                       rows[b], gsems[b])

    def g_wait(i, b):
      pltpu.make_async_copy(table_hbm.at[idx_v.at[pl.ds(i * chunk, chunk)]],
                            rows[b], gsems[b]).wait()

    def w_start(i, b):
      pltpu.async_copy(rows[b],
                       out_hbm.at[pl.ds(base + i * chunk, chunk), pl.ds(0, D)],
                       wsems[b])

    def w_wait(i, b):
      pltpu.make_async_copy(
          rows[b],
          out_hbm.at[pl.ds(base + i * chunk, chunk), pl.ds(0, D)],
          wsems[b]).wait()

    for b in range(_DIST):
      g_start(b, b)

    for b in range(_NBUF):
      i = b
      g_wait(i, b)
      w_start(i, b)
      bj = (b + _DIST) % _NBUF
      if b >= _NBUF - _DIST:
        w_wait(i + _DIST - _NBUF, bj)
      g_start(i + _DIST, bj)

    @pl.loop(1, n_groups - 1)
    def _g(g):
      for b in range(_NBUF):
        i = g * _NBUF + b
        g_wait(i, b)
        w_start(i, b)
        bj = (b + _DIST) % _NBUF
        w_wait(i + _DIST - _NBUF, bj)
        g_start(i + _DIST, bj)

    tail = []
    for b in range(_NBUF):
      i = n_chunks - _NBUF + b
      g_wait(i, b)
      w_start(i, b)
      bj = (b + _DIST) % _NBUF
      w_wait(i + _DIST - _NBUF, bj)
      if b < _NBUF - _DIST:
        g_start(i + _DIST, bj)
      else:
        tail.append((i, b))
    for i, b in tail:
      w_wait(i, b)

  return k


def kernel(x, emb_weight):
  B0, B1 = x.shape
  V, D = emb_weight.shape
  B = B0 * B1
  n_per_w = B // _NW
  chunk = 400

  idx = x.reshape(B).astype(jnp.int32)
  out = _gather_kernel(B, D, n_per_w, chunk)(idx, emb_weight)
  return out[:, :D].reshape(B0, B1, D)


# speedup vs baseline: 1.9713x; 1.0022x over previous
import functools

import jax
import jax.numpy as jnp
from jax import lax
from jax.experimental import pallas as pl
from jax.experimental.pallas import tpu as pltpu
from jax.experimental.pallas import tpu_sc as plsc

_NBUF = 4
_DIST = 2
_NW = 32


def _gather_kernel(B, D, n_per_w, chunk):
  mesh = plsc.VectorSubcoreMesh(core_axis_name="c", subcore_axis_name="s")
  n_chunks = n_per_w // chunk
  n_groups = n_chunks // _NBUF
  assert n_groups >= 2 and n_chunks % _NBUF == 0

  @functools.partial(
      pl.kernel,
      mesh=mesh,
      out_type=jax.ShapeDtypeStruct((B, 128), jnp.float32),
      scratch_types=[
          pltpu.VMEM((n_chunks, chunk), jnp.int32),
          [pltpu.VMEM((chunk, D), jnp.float32)] * _NBUF,
          [pltpu.SemaphoreType.DMA] * _NBUF,
          [pltpu.SemaphoreType.DMA] * _NBUF,
      ],
      compiler_params=pltpu.CompilerParams(use_tc_tiling_on_sc=False),
  )
  def k(idx_hbm, table_hbm, out_hbm, idx_v, rows, gsems, wsems):
    nc = lax.axis_size("c")
    wid = lax.axis_index("s") * nc + lax.axis_index("c")
    base = wid * n_per_w
    pltpu.sync_copy(idx_hbm.at[wid], idx_v)

    def g_start(i, b):
      pltpu.async_copy(table_hbm.at[idx_v.at[i]], rows[b], gsems[b])

    def g_wait(i, b):
      pltpu.make_async_copy(table_hbm.at[idx_v.at[i]], rows[b],
                            gsems[b]).wait()

    def w_start(i, b):
      pltpu.async_copy(rows[b],
                       out_hbm.at[pl.ds(base + i * chunk, chunk), pl.ds(0, D)],
                       wsems[b])

    def w_wait(i, b):
      pltpu.make_async_copy(
          rows[b],
          out_hbm.at[pl.ds(base + i * chunk, chunk), pl.ds(0, D)],
          wsems[b]).wait()

    for b in range(_DIST):
      g_start(b, b)

    for b in range(_NBUF):
      i = b
      g_wait(i, b)
      w_start(i, b)
      bj = (b + _DIST) % _NBUF
      if b >= _NBUF - _DIST:
        w_wait(i + _DIST - _NBUF, bj)
      g_start(i + _DIST, bj)

    @pl.loop(1, n_groups - 1)
    def _g(g):
      for b in range(_NBUF):
        i = g * _NBUF + b
        g_wait(i, b)
        w_start(i, b)
        bj = (b + _DIST) % _NBUF
        w_wait(i + _DIST - _NBUF, bj)
        g_start(i + _DIST, bj)

    tail = []
    for b in range(_NBUF):
      i = n_chunks - _NBUF + b
      g_wait(i, b)
      w_start(i, b)
      bj = (b + _DIST) % _NBUF
      w_wait(i + _DIST - _NBUF, bj)
      if b < _NBUF - _DIST:
        g_start(i + _DIST, bj)
      else:
        tail.append((i, b))
    for i, b in tail:
      w_wait(i, b)

  return k


def kernel(x, emb_weight):
  B0, B1 = x.shape
  V, D = emb_weight.shape
  B = B0 * B1
  n_per_w = B // _NW
  chunk = 400

  idx = x.reshape(_NW, n_per_w // chunk, chunk).astype(jnp.int32)
  out = _gather_kernel(B, D, n_per_w, chunk)(idx, emb_weight)
  return out[:, :D].reshape(B0, B1, D)
